# Initial kernel scaffold; baseline (speedup 1.0000x reference)
import jax
import jax.numpy as jnp
from jax.experimental import pallas as pl


def kernel(x, disc, params):
    raise NotImplementedError("write your pallas kernel here")


# speedup vs baseline: 3.4379x; 3.4379x over previous
import functools

import numpy as np
import jax
import jax.numpy as jnp
from jax import lax
from jax.experimental import pallas as pl
from jax.experimental.pallas import tpu as pltpu
from jax.experimental.pallas import tpu_sc as plsc

KNN = 20
EPS = 1e-5
NB_B = 4
NN = 1024
BN = NB_B * NN
CP = 128
NEG = -3.0e38
HI = lax.Precision.HIGHEST
F32 = jnp.float32
BF16 = jnp.bfloat16


def _dot(a, b, prec=None):
    return lax.dot_general(a, b, (((1,), (0,)), ((), ())), precision=prec,
                           preferred_element_type=F32)


def _dot_nt(a, b, prec=None):
    return lax.dot_general(a, b, (((1,), (1,)), ((), ())), precision=prec,
                           preferred_element_type=F32)


def _lrelu(t):
    return jnp.where(t >= 0, t, 0.2 * t)


def _topk_body(xr_ref, xa_ref, xxr_ref, xxc_ref, idx_ref, *, R):
    b = pl.program_id(0)
    xr = xr_ref[0]
    xa = xa_ref[0]
    xxr = xxr_ref[0]
    xxc = xxc_ref[0]
    G = _dot_nt(xr.astype(BF16), xa.astype(BF16))
    inner = -2.0 * G
    pd = ((-xxr) - inner) - xxc
    iota = lax.broadcasted_iota(jnp.int32, (R, NN), 1)
    ams = []
    for _ in range(KNN):
        m = jnp.max(pd, axis=1, keepdims=True)
        am = jnp.min(jnp.where(pd == m, iota, NN), axis=1, keepdims=True)
        ams.append(am)
        pd = jnp.where(iota == am, NEG, pd)
    idx_ref[0] = jnp.concatenate(ams, axis=1) + b * NN


def _topk_tc(xT, xxr, xxc):
    Bb, Nn, C = xT.shape
    R = 256
    return pl.pallas_call(
        functools.partial(_topk_body, R=R),
        grid=(Bb, Nn // R),
        in_specs=[
            pl.BlockSpec((1, R, C), lambda b, r: (b, r, 0)),
            pl.BlockSpec((1, Nn, C), lambda b, r: (b, 0, 0)),
            pl.BlockSpec((1, R, 1), lambda b, r: (b, r, 0)),
            pl.BlockSpec((1, 1, Nn), lambda b, r: (b, 0, 0)),
        ],
        out_specs=pl.BlockSpec((1, R, KNN), lambda b, r: (b, r, 0)),
        out_shape=jax.ShapeDtypeStruct((Bb, Nn, KNN), jnp.int32),
    )(xT, xT, xxr, xxc)


def _sc_rows(table, idxf):
    E = BN * KNN
    NW = 32
    PE = E // NW
    CH = 512
    NCH = PE // CH
    mesh = plsc.VectorSubcoreMesh(core_axis_name="c", subcore_axis_name="s")

    @functools.partial(
        pl.kernel, mesh=mesh,
        out_type=jax.ShapeDtypeStruct((E, CP), F32),
        scratch_types=[
            pltpu.VMEM((PE,), jnp.int32),
            pltpu.VMEM((CH, CP), F32),
            pltpu.SemaphoreType.DMA,
        ],
    )
    def sck(tab_hbm, idx_hbm, out_hbm, idx_v, buf_v, sem):
        wid = lax.axis_index("s") * 2 + lax.axis_index("c")
        base = wid * PE
        pltpu.sync_copy(idx_hbm.at[pl.ds(base, PE)], idx_v)

        def body(c, carry):
            pltpu.async_copy(
                tab_hbm.at[idx_v.at[pl.ds(c * CH, CH)]], buf_v, sem).wait()
            pltpu.sync_copy(buf_v, out_hbm.at[pl.ds(base + c * CH, CH)])
            return carry

        lax.fori_loop(0, NCH, body, 0)

    return sck(table, idxf)


def _edge_h(feat_ref, xT_ref, wt_ref, R, C):
    gf = feat_ref[0]
    xi = xT_ref[0]
    xib = xi[:, None, :]
    diffb = (gf - xib).astype(BF16)[:, :, :C]
    xkb = jnp.broadcast_to(xib, gf.shape).astype(BF16)[:, :, :C]
    f2 = jnp.concatenate([diffb, xkb], axis=2)
    F2 = f2.reshape(R * KNN, 2 * C)
    return _dot(F2, wt_ref[...].astype(BF16))


def _conva_body(feat_ref, xT_ref, wt_ref, hmax_ref, part_ref, *, R, C, O):
    h = _edge_h(feat_ref, xT_ref, wt_ref, R, C)
    hmax_ref[0] = jnp.max(h.reshape(R, KNN, O), axis=1)
    part_ref[0, 0] = jnp.sum(h, axis=0, keepdims=True)


def _convb_body(feat_ref, xT_ref, wt_ref, m_ref, part_ref, *, R, C, O):
    h = _edge_h(feat_ref, xT_ref, wt_ref, R, C)
    d = h - m_ref[...]
    part_ref[0, 0] = jnp.sum(d * d, axis=0, keepdims=True)


def _conv_a(feat, xT, wt):
    C2, O = wt.shape
    R = 128
    NBK = NN // R
    return pl.pallas_call(
        functools.partial(_conva_body, R=R, C=C2 // 2, O=O),
        grid=(NB_B, NBK),
        in_specs=[
            pl.BlockSpec((1, R, KNN, CP), lambda b, r: (b, r, 0, 0)),
            pl.BlockSpec((1, R, CP), lambda b, r: (b, r, 0)),
            pl.BlockSpec((C2, O), lambda b, r: (0, 0)),
        ],
        out_specs=[
            pl.BlockSpec((1, R, O), lambda b, r: (b, r, 0)),
            pl.BlockSpec((1, 1, 1, O), lambda b, r: (b, r, 0, 0)),
        ],
        out_shape=[
            jax.ShapeDtypeStruct((NB_B, NN, O), F32),
            jax.ShapeDtypeStruct((NB_B, NBK, 1, O), F32),
        ],
    )(feat, xT, wt)


def _conv_b(feat, xT, wt, m):
    C2, O = wt.shape
    R = 128
    NBK = NN // R
    return pl.pallas_call(
        functools.partial(_convb_body, R=R, C=C2 // 2, O=O),
        grid=(NB_B, NBK),
        in_specs=[
            pl.BlockSpec((1, R, KNN, CP), lambda b, r: (b, r, 0, 0)),
            pl.BlockSpec((1, R, CP), lambda b, r: (b, r, 0)),
            pl.BlockSpec((C2, O), lambda b, r: (0, 0)),
            pl.BlockSpec((1, O), lambda b, r: (0, 0)),
        ],
        out_specs=pl.BlockSpec((1, 1, 1, O), lambda b, r: (b, r, 0, 0)),
        out_shape=jax.ShapeDtypeStruct((NB_B, NBK, 1, O), F32),
    )(feat, xT, wt, m)


def _mean_body(part_ref, m_ref):
    cnt = np.float32(BN * KNN)
    m_ref[...] = jnp.sum(part_ref[:, 0, :], axis=0, keepdims=True) / cnt


def _mean(parts):
    M, _, O = parts.shape
    return pl.pallas_call(
        _mean_body,
        out_shape=jax.ShapeDtypeStruct((1, O), F32),
    )(parts)


def _combine_body(mx_ref, part_ref, m_ref, g_ref, b_ref, o_ref):
    cnt = np.float32(BN * KNN)
    var = jnp.sum(part_ref[:, 0, :], axis=0, keepdims=True) / cnt
    xn = (mx_ref[...] - m_ref[...]) / jnp.sqrt(var + EPS)
    t = xn * g_ref[...] + b_ref[...]
    o_ref[...] = _lrelu(t)


def _combine(mx, partsb, m, g, b):
    O = mx.shape[1]
    M = partsb.shape[0]
    CB = min(O, 128)
    return pl.pallas_call(
        _combine_body,
        grid=(O // CB,),
        in_specs=[
            pl.BlockSpec((BN, CB), lambda i: (0, i)),
            pl.BlockSpec((M, 1, CB), lambda i: (0, 0, i)),
            pl.BlockSpec((1, CB), lambda i: (0, i)),
            pl.BlockSpec((1, CB), lambda i: (0, i)),
            pl.BlockSpec((1, CB), lambda i: (0, i)),
        ],
        out_specs=pl.BlockSpec((BN, CB), lambda i: (0, i)),
        out_shape=jax.ShapeDtypeStruct((BN, O), F32),
    )(mx, partsb, m, g, b)


def _bn_rows(t, g, b):
    m = jnp.mean(t, axis=0, keepdims=True)
    v = jnp.mean((t - m) * (t - m), axis=0, keepdims=True)
    return (t - m) / jnp.sqrt(v + EPS) * g + b


def _bdot(a, b):
    return _dot(a.astype(BF16), b.astype(BF16))


def _erf(t):
    at = jnp.abs(t)
    s = jnp.where(t >= 0, 1.0, -1.0)
    w = 1.0 / (1.0 + 0.3275911 * at)
    poly = ((((1.061405429 * w - 1.453152027) * w + 1.421413741) * w
             - 0.284496736) * w + 0.254829592) * w
    return s * (1.0 - poly * jnp.exp(-at * at))


def _head_body(cat_ref, w5_ref, g5_ref, b5_ref,
               l1_ref, g6_ref, b6_ref, l2_ref, g7_ref, b7_ref,
               l21_ref, l22_ref, g8_ref, b8_ref, l3_ref,
               out_ref, y_ref):
    w5b = w5_ref[...].astype(BF16)
    ssum = jnp.zeros((1, 1024), F32)
    for b in range(NB_B):
        y = _dot(cat_ref[b].astype(BF16), w5b)
        y_ref[b] = y
        ssum = ssum + jnp.sum(y, axis=0, keepdims=True)
    cnt = np.float32(BN)
    mean = ssum / cnt
    vsum = jnp.zeros((1, 1024), F32)
    for b in range(NB_B):
        dy = y_ref[b] - mean
        vsum = vsum + jnp.sum(dy * dy, axis=0, keepdims=True)
    var = vsum / cnt
    rs = jnp.sqrt(var + EPS)
    zrows = []
    for b in range(NB_B):
        h = _lrelu((y_ref[b] - mean) / rs * g5_ref[...] + b5_ref[...])
        zmax = jnp.max(h, axis=0, keepdims=True)
        zavg = jnp.sum(h, axis=0, keepdims=True) * (1.0 / float(NN))
        zrows.append(jnp.concatenate([zmax, zavg], axis=1))
    z = jnp.concatenate(zrows, axis=0)
    z = _lrelu(_bn_rows(_bdot(z, l1_ref[...]), g6_ref[...], b6_ref[...]))
    z = _lrelu(_bn_rows(_bdot(z, l2_ref[...]), g7_ref[...], b7_ref[...]))
    z = _bdot(z, l21_ref[...])
    z = _bdot(z, l22_ref[...])
    zb = _bn_rows(z, g8_ref[...], b8_ref[...])
    ge = zb * 0.5 * (1.0 + _erf(zb * np.float32(1.0 / np.sqrt(2.0))))
    out_ref[...] = _bdot(ge, l3_ref[...])


def _head(cat, w5t, g5, b5, l1, g6, b6, l2, g7, b7, l21, l22, g8, b8, l3):
    return pl.pallas_call(
        _head_body,
        out_shape=jax.ShapeDtypeStruct((NB_B, 90), F32),
        scratch_shapes=[pltpu.VMEM((NB_B, NN, 1024), F32)],
    )(cat, w5t, g5, b5, l1, g6, b6, l2, g7, b7, l21, l22, g8, b8, l3)


def kernel(x, disc, params):
    p = params
    xT = jnp.transpose(x, (0, 2, 1))
    xT = jnp.pad(xT, ((0, 0), (0, 0), (0, CP - 6)))

    feats = []
    cur = xT
    xc = x
    cs = [6, 64, 64, 128]
    for li in range(4):
        W = p['conv%d_w' % (li + 1)]
        C = cs[li]
        O = W.shape[0]
        wt = W.T
        xx = jnp.sum(xc ** 2, axis=1, keepdims=True)
        xxr = jnp.transpose(xx, (0, 2, 1))
        idx = _topk_tc(cur, xxr, xx)
        rows = _sc_rows(cur.reshape(BN, CP), idx.reshape(BN * KNN))
        feat = rows.reshape(NB_B, NN, KNN, CP)
        hmax, parts_a = _conv_a(feat, cur, wt)
        m = _mean(parts_a.reshape(-1, 1, O))
        parts_b = _conv_b(feat, cur, wt, m)
        xn = _combine(hmax.reshape(BN, O), parts_b.reshape(-1, 1, O), m,
                      p['bn%d_g' % (li + 1)].reshape(1, O),
                      p['bn%d_b' % (li + 1)].reshape(1, O))
        xl = xn.reshape(NB_B, NN, O)
        feats.append(xl)
        if li < 3:
            cur = jnp.pad(xn, ((0, 0), (0, CP - O))).reshape(NB_B, NN, CP)
            xc = lax.optimization_barrier(jnp.transpose(xl, (0, 2, 1)))

    cat = jnp.concatenate(feats, axis=2)
    out = _head(
        cat, p['conv5_w'].T,
        p['bn5_g'].reshape(1, 1024), p['bn5_b'].reshape(1, 1024),
        p['lin1_w'].T, p['bn6_g'].reshape(1, 256), p['bn6_b'].reshape(1, 256),
        p['lin2_w'].T, p['bn7_g'].reshape(1, 128), p['bn7_b'].reshape(1, 128),
        p['lin21_w'].T, p['lin22_w'].T,
        p['bn8_g'].reshape(1, 32), p['bn8_b'].reshape(1, 32),
        p['lin3_w'].T)
    return out
